# baseline (speedup 1.0000x reference)
import functools

import jax
import jax.numpy as jnp
from jax import lax
from jax.experimental import pallas as pl
from jax.experimental.pallas import tpu as pltpu
from jax.experimental.pallas import tpu_sc as plsc

VOCAB = 1000000
BATCH = 4096
HIST = 200
MODEL_DIM = 64
SROW = 2 * MODEL_DIM
N = BATCH * HIST
NC, NS = 2, 16
NW = NC * NS
N_PER_W = N // NW
GATHER_I = 128
K = 4
CHUNK = K * GATHER_I
T = N_PER_W // CHUNK

SEL_R = 1024


def _emb_gather_sc(idx2d, tab2):
    mesh = plsc.VectorSubcoreMesh(
        core_axis_name="c", subcore_axis_name="s", num_cores=NC, num_subcores=NS
    )

    @functools.partial(
        pl.kernel,
        out_type=jax.ShapeDtypeStruct((N, SROW), jnp.float32),
        mesh=mesh,
        scratch_types=[
            pltpu.VMEM((K, GATHER_I), jnp.int32),
            pltpu.VMEM((CHUNK, SROW), jnp.float32),
            pltpu.SemaphoreType.DMA,
        ],
    )
    def body(idx_hbm, tab_hbm, out_hbm, idx_v, rows_v, gsem):
        wid = lax.axis_index("s") * NC + lax.axis_index("c")
        row0 = wid * (N_PER_W // GATHER_I)
        out0 = wid * N_PER_W

        @pl.loop(0, T)
        def _chunk(t):
            pltpu.sync_copy(idx_hbm.at[pl.ds(row0 + t * K, K)], idx_v)
            copies = [
                pltpu.async_copy(
                    tab_hbm.at[idx_v.at[j]],
                    rows_v.at[pl.ds(j * GATHER_I, GATHER_I)],
                    gsem,
                )
                for j in range(K)
            ]
            for cp in copies:
                cp.wait()
            pltpu.sync_copy(rows_v, out_hbm.at[pl.ds(out0 + t * CHUNK, CHUNK)])

    return body(idx2d, tab2)


def _half_select_tc(srows, parity):
    def body(rows_ref, par_ref, out_ref):
        rows = rows_ref[...]
        odd = par_ref[...] == 1
        out_ref[...] = jnp.where(odd, rows[:, MODEL_DIM:], rows[:, :MODEL_DIM])

    return pl.pallas_call(
        body,
        grid=(N // SEL_R,),
        in_specs=[
            pl.BlockSpec((SEL_R, SROW), lambda i: (i, 0)),
            pl.BlockSpec((SEL_R, 1), lambda i: (i, 0)),
        ],
        out_specs=pl.BlockSpec((SEL_R, MODEL_DIM), lambda i: (i, 0)),
        out_shape=jax.ShapeDtypeStruct((N, MODEL_DIM), jnp.float32),
    )(srows, parity)


def _mask_tc(word_idx):
    def body(idx_ref, out_ref):
        out_ref[...] = idx_ref[...] > 0

    return pl.pallas_call(
        body,
        out_shape=jax.ShapeDtypeStruct((BATCH, HIST), jnp.bool_),
    )(word_idx)


def kernel(word_idx, emb_table):
    flat_idx = word_idx.reshape(N)
    sidx2d = (flat_idx >> 1).reshape(N // GATHER_I, GATHER_I)
    parity = (flat_idx & 1).reshape(N, 1)
    tab2 = emb_table.reshape(VOCAB // 2, SROW)
    srows = _emb_gather_sc(sidx2d, tab2)
    emb_flat = _half_select_tc(srows, parity)
    static_emb = emb_flat.reshape(BATCH, HIST, MODEL_DIM)
    bottom_existence = _mask_tc(word_idx)
    return (static_emb, bottom_existence)


# speedup vs baseline: 1.2383x; 1.2383x over previous
import functools

import jax
import jax.numpy as jnp
from jax import lax
from jax.experimental import pallas as pl
from jax.experimental.pallas import tpu as pltpu
from jax.experimental.pallas import tpu_sc as plsc

VOCAB = 1000000
BATCH = 4096
HIST = 200
MODEL_DIM = 64
SROW = 2 * MODEL_DIM
N = BATCH * HIST
NC, NS = 2, 16
NW = NC * NS
N_PER_W = N // NW
GATHER_I = 128
IDX_ROWS = N_PER_W // GATHER_I
K = 2
CHUNK = K * GATHER_I
T = N_PER_W // CHUNK

SEL_R = 8192


def _emb_gather_sc(idx2d, tab2):
    mesh = plsc.VectorSubcoreMesh(
        core_axis_name="c", subcore_axis_name="s", num_cores=NC, num_subcores=NS
    )

    @functools.partial(
        pl.kernel,
        out_type=jax.ShapeDtypeStruct((N, SROW), jnp.float32),
        mesh=mesh,
        scratch_types=[
            pltpu.VMEM((IDX_ROWS, GATHER_I), jnp.int32),
            pltpu.VMEM((CHUNK, SROW), jnp.float32),
            pltpu.VMEM((CHUNK, SROW), jnp.float32),
            pltpu.SemaphoreType.DMA,
            pltpu.SemaphoreType.DMA,
        ],
    )
    def body(idx_hbm, tab_hbm, out_hbm, idx_v, rows0, rows1, sem0, sem1):
        rows_v = (rows0, rows1)
        gsem = (sem0, sem1)
        wid = lax.axis_index("s") * NC + lax.axis_index("c")
        row0 = wid * IDX_ROWS
        out0 = wid * N_PER_W

        pltpu.sync_copy(idx_hbm.at[pl.ds(row0, IDX_ROWS)], idx_v)

        def fire(c, b):
            for j in range(K):
                pltpu.async_copy(
                    tab_hbm.at[idx_v.at[c * K + j]],
                    rows_v[b].at[pl.ds(j * GATHER_I, GATHER_I)],
                    gsem[b],
                )

        for b in range(2):
            fire(b, b)

        @pl.loop(0, T // 2)
        def _round(g):
            for b in range(2):
                c = 2 * g + b
                pltpu.make_async_copy(
                    tab_hbm.at[pl.ds(0, CHUNK)], rows_v[b], gsem[b]
                ).wait()
                pltpu.sync_copy(
                    rows_v[b], out_hbm.at[pl.ds(out0 + c * CHUNK, CHUNK)]
                )

                @pl.when(c + 2 < T)
                def _():
                    fire(c + 2, b)

    return body(idx2d, tab2)


def _half_select_tc(srows, parity):
    def body(rows_ref, par_ref, out_ref):
        rows = rows_ref[...]
        odd = par_ref[...] == 1
        out_ref[...] = jnp.where(odd, rows[:, MODEL_DIM:], rows[:, :MODEL_DIM])

    return pl.pallas_call(
        body,
        grid=(N // SEL_R,),
        in_specs=[
            pl.BlockSpec((SEL_R, SROW), lambda i: (i, 0)),
            pl.BlockSpec((SEL_R, 1), lambda i: (i, 0)),
        ],
        out_specs=pl.BlockSpec((SEL_R, MODEL_DIM), lambda i: (i, 0)),
        out_shape=jax.ShapeDtypeStruct((N, MODEL_DIM), jnp.float32),
    )(srows, parity)


def _mask_tc(word_idx):
    def body(idx_ref, out_ref):
        out_ref[...] = idx_ref[...] > 0

    return pl.pallas_call(
        body,
        out_shape=jax.ShapeDtypeStruct((BATCH, HIST), jnp.bool_),
    )(word_idx)


def kernel(word_idx, emb_table):
    flat_idx = word_idx.reshape(N)
    sidx2d = (flat_idx >> 1).reshape(N // GATHER_I, GATHER_I)
    parity = (flat_idx & 1).reshape(N, 1)
    tab2 = emb_table.reshape(VOCAB // 2, SROW)
    srows = _emb_gather_sc(sidx2d, tab2)
    emb_flat = _half_select_tc(srows, parity)
    static_emb = emb_flat.reshape(BATCH, HIST, MODEL_DIM)
    bottom_existence = _mask_tc(word_idx)
    return (static_emb, bottom_existence)
